# baseline (speedup 1.0000x reference)
import functools

import jax
import jax.numpy as jnp
from jax import lax
from jax.experimental import pallas as pl
from jax.experimental.pallas import tpu as pltpu
from jax.experimental.pallas import tpu_sc as plsc

E = 64
D = 1024
H = 4096
TOPK = 2
BALANCE_COEFF = 0.01
N_TOK = 32
HT = 1024
NH = H // HT
L = 16
NCH = E // L


def _logits_kernel(x_ref, wg_ref, out_ref):
    out_ref[...] = jax.lax.dot_general(
        x_ref[...], wg_ref[...], (((1,), (1,)), ((), ())),
        preferred_element_type=jnp.float32)


def _route_kernel(logits_hbm, g_hbm, meta_hbm, aux_hbm,
                  logits_v, g_v, ids_v, row_v):
    cid = lax.axis_index("c")
    sid = lax.axis_index("s")

    @pl.when((cid == 0) & (sid == 0))
    def _run():
        pltpu.sync_copy(logits_hbm, logits_v)
        lanes = jax.lax.broadcasted_iota(jnp.int32, (L,), 0)
        zeros = jnp.zeros((L,), jnp.float32)

        def token_body(t, counts):
            chunks = [logits_v[t, pl.ds(c * L, L)] for c in range(NCH)]
            m1 = jnp.max(chunks[0])
            for c in range(1, NCH):
                m1 = jnp.maximum(m1, jnp.max(chunks[c]))
            idx1 = jnp.int32(E)
            for c in range(NCH):
                cand = jnp.min(jnp.where(chunks[c] == m1, lanes + c * L, E))
                idx1 = jnp.minimum(idx1, cand)
            masked = [jnp.where(lanes + c * L == idx1, -1e30, chunks[c])
                      for c in range(NCH)]
            m2 = jnp.max(masked[0])
            for c in range(1, NCH):
                m2 = jnp.maximum(m2, jnp.max(masked[c]))
            idx2 = jnp.int32(E)
            for c in range(NCH):
                cand = jnp.min(jnp.where(masked[c] == m2, lanes + c * L, E))
                idx2 = jnp.minimum(idx2, cand)
            s = jnp.exp(jnp.full((L,), m2 - m1, jnp.float32))
            g1 = 1.0 / (1.0 + s)
            g2 = s / (1.0 + s)
            new_counts = []
            for c in range(NCH):
                eids = lanes + c * L
                sel1 = eids == idx1
                sel2 = eids == idx2
                g_v[t, pl.ds(c * L, L)] = (jnp.where(sel1, g1, zeros)
                                           + jnp.where(sel2, g2, zeros))
                new_counts.append(counts[c]
                                  + jnp.where(sel1, 1.0, 0.0)
                                  + jnp.where(sel2, 1.0, 0.0))
            return tuple(new_counts)

        counts = lax.fori_loop(0, N_TOK, token_body,
                               tuple(zeros for _ in range(NCH)))
        pltpu.sync_copy(g_v, g_hbm)

        aux = jnp.float32(0.0)
        for c in range(NCH):
            load = counts[c] * (1.0 / (N_TOK * TOPK))
            aux = aux + jnp.sum(load * load)
        aux = BALANCE_COEFF * (E * aux)
        row_v[pl.ds(0, L)] = jnp.where(lanes == 0, jnp.full((L,), aux), zeros)
        pltpu.sync_copy(row_v, aux_hbm)

        sel = [counts[c] > 0.0 for c in range(NCH)]
        selF = [jnp.where(sel[c], 1.0, 0.0) for c in range(NCH)]
        nsel = [jnp.sum(selF[c]) for c in range(NCH)]
        off = jnp.float32(0.0)
        ranks = []
        for c in range(NCH):
            ranks.append(plsc.cumsum(selF[c]) - selF[c] + off)
            off = off + nsel[c]
        n = off
        last_id = jnp.float32(0.0)
        for c in range(NCH):
            eidsF = (lanes + c * L).astype(jnp.float32)
            hit = sel[c] & (ranks[c] == n - 1.0)
            last_id = last_id + jnp.sum(jnp.where(hit, eidsF, 0.0))
        for c in range(NCH):
            plsc.store_scatter(ids_v, [ranks[c].astype(jnp.int32)],
                               lanes + c * L, mask=sel[c])
        n_i = n.astype(jnp.int32)
        last_i = last_id.astype(jnp.int32)
        for c in range(NCH):
            pos = lanes + c * L
            vec = ids_v[pl.ds(c * L, L)]
            ids_v[pl.ds(c * L, L)] = jnp.where(pos < n_i, vec, last_i)
        izeros = jnp.zeros((L,), jnp.int32)
        for c in range(NCH, 8):
            pos = lanes + c * L
            ids_v[pl.ds(c * L, L)] = jnp.where(pos == E, n_i, izeros)
        pltpu.sync_copy(ids_v, meta_hbm)


def _ffn_kernel(meta_ref, x_ref, g_ref, w1_ref, b1_ref, w2_ref, b2_ref,
                out_ref):
    i = pl.program_id(0)
    h = pl.program_id(1)

    @pl.when((i == 0) & (h == 0))
    def _init():
        out_ref[...] = jnp.zeros_like(out_ref)

    n = meta_ref[E]

    @pl.when(i < n)
    def _body():
        e = meta_ref[i]
        lane = jax.lax.broadcasted_iota(jnp.int32, (N_TOK, E), 1)
        w = jnp.sum(jnp.where(lane == e, g_ref[...], 0.0),
                    axis=1, keepdims=True)
        hp = jax.lax.dot_general(
            x_ref[...].astype(jnp.bfloat16),
            w1_ref[0].astype(jnp.bfloat16), (((1,), (1,)), ((), ())),
            preferred_element_type=jnp.float32)
        hp = hp + b1_ref[0, :, pl.ds(h * HT, HT)]
        hp = hp * jax.nn.sigmoid(hp)
        yp = jax.lax.dot_general(
            hp.astype(jnp.bfloat16),
            w2_ref[0].astype(jnp.bfloat16), (((1,), (1,)), ((), ())),
            preferred_element_type=jnp.float32)
        yp = yp + jnp.where(h == 0, 1.0, 0.0) * b2_ref[0]
        out_ref[...] += w * yp


@jax.jit
def kernel(x, Wg, W1, b1, W2, b2):
    orig_shape = x.shape
    x_flat = x.reshape(-1, x.shape[-1])

    logits = pl.pallas_call(
        _logits_kernel,
        out_shape=jax.ShapeDtypeStruct((N_TOK, E), jnp.float32),
    )(x_flat, Wg)

    route = pl.kernel(
        _route_kernel,
        out_type=(
            jax.ShapeDtypeStruct((N_TOK, E), jnp.float32),
            jax.ShapeDtypeStruct((128,), jnp.int32),
            jax.ShapeDtypeStruct((L,), jnp.float32),
        ),
        mesh=plsc.VectorSubcoreMesh(core_axis_name="c", subcore_axis_name="s",
                                    num_cores=2, num_subcores=16),
        scratch_types=[
            pltpu.VMEM((N_TOK, E), jnp.float32),
            pltpu.VMEM((N_TOK, E), jnp.float32),
            pltpu.VMEM((128,), jnp.int32),
            pltpu.VMEM((L,), jnp.float32),
        ],
        compiler_params=pltpu.CompilerParams(needs_layout_passes=False),
    )
    G, meta, aux = route(logits)

    b1r = b1.reshape(E, 1, H)
    b2r = b2.reshape(E, 1, D)

    def _w1_map(i, h, m):
        pad = i >= m[E]
        return (m[i], jnp.where(pad, NH - 1, h), 0)

    def _w2_map(i, h, m):
        pad = i >= m[E]
        return (m[i], 0, jnp.where(pad, NH - 1, h))

    def _eb_map(i, h, m):
        return (m[i], 0, 0)

    def _const_map(i, h, m):
        return (0, 0)

    grid_spec = pltpu.PrefetchScalarGridSpec(
        num_scalar_prefetch=1,
        grid=(E, NH),
        in_specs=[
            pl.BlockSpec((N_TOK, D), _const_map),
            pl.BlockSpec((N_TOK, E), _const_map),
            pl.BlockSpec((1, HT, D), _w1_map),
            pl.BlockSpec((1, 1, H), _eb_map),
            pl.BlockSpec((1, D, HT), _w2_map),
            pl.BlockSpec((1, 1, D), _eb_map),
        ],
        out_specs=pl.BlockSpec((N_TOK, D), _const_map),
    )

    out = pl.pallas_call(
        _ffn_kernel,
        grid_spec=grid_spec,
        out_shape=jax.ShapeDtypeStruct((N_TOK, D), jnp.float32),
        compiler_params=pltpu.CompilerParams(
            dimension_semantics=("arbitrary", "arbitrary")),
    )(meta, x_flat, G, W1, b1r, W2, b2r)

    output = out.reshape(orig_shape)
    aux_loss = jnp.reshape(aux[0:1], ())
    return output, aux_loss


# speedup vs baseline: 1.0056x; 1.0056x over previous
import jax
import jax.numpy as jnp
from jax import lax
from jax.experimental import pallas as pl
from jax.experimental.pallas import tpu as pltpu
from jax.experimental.pallas import tpu_sc as plsc

E = 64
D = 1024
H = 4096
TOPK = 2
BALANCE_COEFF = 0.01
N_TOK = 32
HT = 1024
NH = H // HT
L = 16
NCH = E // L


def _logits_kernel(x_ref, wg_ref, out_ref):
    out_ref[...] = jax.lax.dot_general(
        x_ref[...], wg_ref[...], (((1,), (1,)), ((), ())),
        preferred_element_type=jnp.float32)


def _route_kernel(logits_hbm, g_hbm, meta_hbm, aux_hbm,
                  logits_v, g_v, ids_v, row_v):
    cid = lax.axis_index("c")
    sid = lax.axis_index("s")

    @pl.when((cid == 0) & (sid == 0))
    def _run():
        pltpu.sync_copy(logits_hbm, logits_v)
        lanes = jax.lax.broadcasted_iota(jnp.int32, (L,), 0)
        zeros = jnp.zeros((L,), jnp.float32)

        def token_body(t, counts):
            chunks = [logits_v[t, pl.ds(c * L, L)] for c in range(NCH)]
            m1 = jnp.max(chunks[0])
            for c in range(1, NCH):
                m1 = jnp.maximum(m1, jnp.max(chunks[c]))
            idx1 = jnp.int32(E)
            for c in range(NCH):
                cand = jnp.min(jnp.where(chunks[c] == m1, lanes + c * L, E))
                idx1 = jnp.minimum(idx1, cand)
            masked = [jnp.where(lanes + c * L == idx1, -1e30, chunks[c])
                      for c in range(NCH)]
            m2 = jnp.max(masked[0])
            for c in range(1, NCH):
                m2 = jnp.maximum(m2, jnp.max(masked[c]))
            idx2 = jnp.int32(E)
            for c in range(NCH):
                cand = jnp.min(jnp.where(masked[c] == m2, lanes + c * L, E))
                idx2 = jnp.minimum(idx2, cand)
            s = jnp.exp(jnp.full((L,), m2 - m1, jnp.float32))
            g1 = 1.0 / (1.0 + s)
            g2 = s / (1.0 + s)
            new_counts = []
            for c in range(NCH):
                eids = lanes + c * L
                sel1 = eids == idx1
                sel2 = eids == idx2
                g_v[t, pl.ds(c * L, L)] = (jnp.where(sel1, g1, zeros)
                                           + jnp.where(sel2, g2, zeros))
                new_counts.append(counts[c]
                                  + jnp.where(sel1, 1.0, 0.0)
                                  + jnp.where(sel2, 1.0, 0.0))
            return tuple(new_counts)

        counts = lax.fori_loop(0, N_TOK, token_body,
                               tuple(zeros for _ in range(NCH)))
        pltpu.sync_copy(g_v, g_hbm)

        aux = jnp.float32(0.0)
        for c in range(NCH):
            load = counts[c] * (1.0 / (N_TOK * TOPK))
            aux = aux + jnp.sum(load * load)
        aux = BALANCE_COEFF * (E * aux)
        row_v[pl.ds(0, L)] = jnp.where(lanes == 0, jnp.full((L,), aux), zeros)
        pltpu.sync_copy(row_v, aux_hbm)

        sel = [counts[c] > 0.0 for c in range(NCH)]
        selF = [jnp.where(sel[c], 1.0, 0.0) for c in range(NCH)]
        nsel = [jnp.sum(selF[c]) for c in range(NCH)]
        off = jnp.float32(0.0)
        ranks = []
        for c in range(NCH):
            ranks.append(plsc.cumsum(selF[c]) - selF[c] + off)
            off = off + nsel[c]
        n = off
        last_id = jnp.float32(0.0)
        for c in range(NCH):
            eidsF = (lanes + c * L).astype(jnp.float32)
            hit = sel[c] & (ranks[c] == n - 1.0)
            last_id = last_id + jnp.sum(jnp.where(hit, eidsF, 0.0))
        for c in range(NCH):
            plsc.store_scatter(ids_v, [ranks[c].astype(jnp.int32)],
                               lanes + c * L, mask=sel[c])
        n_i = n.astype(jnp.int32)
        last_i = last_id.astype(jnp.int32)
        for c in range(NCH):
            pos = lanes + c * L
            vec = ids_v[pl.ds(c * L, L)]
            ids_v[pl.ds(c * L, L)] = jnp.where(pos < n_i, vec, last_i)
        izeros = jnp.zeros((L,), jnp.int32)
        for c in range(NCH, 8):
            pos = lanes + c * L
            ids_v[pl.ds(c * L, L)] = jnp.where(pos == E, n_i, izeros)
        pltpu.sync_copy(ids_v, meta_hbm)


def _ffn_kernel(meta_ref, x_ref, g_ref, w1_ref, b1_ref, w2_ref, b2_ref,
                out_ref):
    i = pl.program_id(0)
    h = pl.program_id(1)

    @pl.when((i == 0) & (h == 0))
    def _init():
        out_ref[...] = jnp.zeros_like(out_ref)

    n = meta_ref[E]

    @pl.when(i < n)
    def _body():
        e = meta_ref[i]
        lane = jax.lax.broadcasted_iota(jnp.int32, (N_TOK, E), 1)
        w = jnp.sum(jnp.where(lane == e, g_ref[...], 0.0),
                    axis=1, keepdims=True)
        hp = jax.lax.dot_general(
            x_ref[...], w1_ref[0], (((1,), (1,)), ((), ())),
            preferred_element_type=jnp.float32)
        hp = hp + b1_ref[0, :, pl.ds(h * HT, HT)]
        hp = hp * jax.nn.sigmoid(hp)
        yp = jax.lax.dot_general(
            hp, w2_ref[0], (((1,), (1,)), ((), ())),
            preferred_element_type=jnp.float32)
        yp = yp + jnp.where(h == 0, 1.0, 0.0) * b2_ref[0]
        out_ref[...] += w * yp


@jax.jit
def kernel(x, Wg, W1, b1, W2, b2):
    orig_shape = x.shape
    x_flat = x.reshape(-1, x.shape[-1])

    logits = pl.pallas_call(
        _logits_kernel,
        out_shape=jax.ShapeDtypeStruct((N_TOK, E), jnp.float32),
    )(x_flat, Wg)

    route = pl.kernel(
        _route_kernel,
        out_type=(
            jax.ShapeDtypeStruct((N_TOK, E), jnp.float32),
            jax.ShapeDtypeStruct((128,), jnp.int32),
            jax.ShapeDtypeStruct((L,), jnp.float32),
        ),
        mesh=plsc.VectorSubcoreMesh(core_axis_name="c", subcore_axis_name="s",
                                    num_cores=2, num_subcores=16),
        scratch_types=[
            pltpu.VMEM((N_TOK, E), jnp.float32),
            pltpu.VMEM((N_TOK, E), jnp.float32),
            pltpu.VMEM((128,), jnp.int32),
            pltpu.VMEM((L,), jnp.float32),
        ],
        compiler_params=pltpu.CompilerParams(needs_layout_passes=False),
    )
    G, meta, aux = route(logits)

    b1r = b1.reshape(E, 1, H)
    b2r = b2.reshape(E, 1, D)

    def _w1_map(i, h, m):
        pad = i >= m[E]
        return (m[i], jnp.where(pad, NH - 1, h), 0)

    def _w2_map(i, h, m):
        pad = i >= m[E]
        return (m[i], 0, jnp.where(pad, NH - 1, h))

    def _eb_map(i, h, m):
        return (m[i], 0, 0)

    def _const_map(i, h, m):
        return (0, 0)

    grid_spec = pltpu.PrefetchScalarGridSpec(
        num_scalar_prefetch=1,
        grid=(E, NH),
        in_specs=[
            pl.BlockSpec((N_TOK, D), _const_map),
            pl.BlockSpec((N_TOK, E), _const_map),
            pl.BlockSpec((1, HT, D), _w1_map),
            pl.BlockSpec((1, 1, H), _eb_map),
            pl.BlockSpec((1, D, HT), _w2_map),
            pl.BlockSpec((1, 1, D), _eb_map),
        ],
        out_specs=pl.BlockSpec((N_TOK, D), _const_map),
    )

    out = pl.pallas_call(
        _ffn_kernel,
        grid_spec=grid_spec,
        out_shape=jax.ShapeDtypeStruct((N_TOK, D), jnp.float32),
        compiler_params=pltpu.CompilerParams(
            dimension_semantics=("arbitrary", "arbitrary")),
    )(meta, x_flat, G, W1, b1r, W2, b2r)

    output = out.reshape(orig_shape)
    aux_loss = jnp.reshape(aux[0:1], ())
    return output, aux_loss
